# baseline (speedup 1.0000x reference)
import dataclasses

import jax
import jax.numpy as jnp
from jax import lax
from jax.experimental import pallas as pl
from jax.experimental.pallas import tpu as pltpu
from jax.experimental.pallas import tpu_sc as plsc

N_USERS = 1000000
N_ITEMS = 100000
N_FACTORS = 32
BATCH = 16384

NUM_CORES = 2
NUM_SUBCORES = 16
NUM_WORKERS = NUM_CORES * NUM_SUBCORES
B_PER_W = BATCH // NUM_WORKERS
IDX_CHUNK = 128
CHUNKS_PER_W = B_PER_W // IDX_CHUNK
LANES = 16
VECS_PER_CHUNK = IDX_CHUNK // LANES


def _body(users_hbm, items_hbm, u_hbm, v_hbm, out_hbm,
          uidx, vidx, uoff, voff, u_g, v_g, out_v, sem):
  wid = lax.axis_index("s") * NUM_CORES + lax.axis_index("c")

  pltpu.sync_copy(users_hbm.at[pl.ds(wid * CHUNKS_PER_W, CHUNKS_PER_W)], uidx)
  pltpu.sync_copy(items_hbm.at[pl.ds(wid * CHUNKS_PER_W, CHUNKS_PER_W)], vidx)

  for c in range(CHUNKS_PER_W):
    for i in range(VECS_PER_CHUNK):
      s = pl.ds(i * LANES, LANES)
      uq = uidx.at[c][s]
      vq = vidx.at[c][s]
      for k in range(N_FACTORS):
        uoff.at[c, k][s] = uq + (k * N_USERS)
        voff.at[c, k][s] = vq + (k * N_ITEMS)

  copies = []
  for c in range(CHUNKS_PER_W):
    for k in range(N_FACTORS):
      copies.append(pltpu.async_copy(
          u_hbm.at[uoff.at[c, k]],
          u_g.at[k, pl.ds(c * IDX_CHUNK, IDX_CHUNK)], sem))
      copies.append(pltpu.async_copy(
          v_hbm.at[voff.at[c, k]],
          v_g.at[k, pl.ds(c * IDX_CHUNK, IDX_CHUNK)], sem))
  for cp in copies:
    cp.wait()

  @pl.loop(0, B_PER_W, step=LANES)
  def _(j):
    s = pl.ds(j, LANES)
    acc = u_g.at[0][s] * v_g.at[0][s]
    for k in range(1, N_FACTORS):
      acc = acc + u_g.at[k][s] * v_g.at[k][s]
    out_v[s] = acc

  pltpu.sync_copy(out_v, out_hbm.at[pl.ds(wid * B_PER_W, B_PER_W)])


@jax.jit
def _pmf_sc(users, items, u_flat, v_flat):
  mesh = plsc.VectorSubcoreMesh(
      core_axis_name="c", subcore_axis_name="s",
      num_cores=NUM_CORES, num_subcores=NUM_SUBCORES)
  cp = pltpu.CompilerParams(use_tc_tiling_on_sc=False)
  if "needs_layout_passes" in pltpu.CompilerParams.__dataclass_fields__:
    cp = dataclasses.replace(cp, needs_layout_passes=False)
  run = pl.kernel(
      _body,
      out_type=jax.ShapeDtypeStruct((BATCH,), jnp.float32),
      mesh=mesh,
      compiler_params=cp,
      scratch_types=[
          pltpu.VMEM((CHUNKS_PER_W, IDX_CHUNK), jnp.int32),
          pltpu.VMEM((CHUNKS_PER_W, IDX_CHUNK), jnp.int32),
          pltpu.VMEM((CHUNKS_PER_W, N_FACTORS, IDX_CHUNK), jnp.int32),
          pltpu.VMEM((CHUNKS_PER_W, N_FACTORS, IDX_CHUNK), jnp.int32),
          pltpu.VMEM((N_FACTORS, B_PER_W), jnp.float32),
          pltpu.VMEM((N_FACTORS, B_PER_W), jnp.float32),
          pltpu.VMEM((B_PER_W,), jnp.float32),
          pltpu.SemaphoreType.DMA,
      ],
  )
  return run(users, items, u_flat, v_flat)


def kernel(users_index, items_index, U, V):
  users = users_index.astype(jnp.int32).reshape(BATCH // IDX_CHUNK, IDX_CHUNK)
  items = items_index.astype(jnp.int32).reshape(BATCH // IDX_CHUNK, IDX_CHUNK)
  u_flat = U.T.reshape(N_USERS * N_FACTORS)
  v_flat = V.T.reshape(N_ITEMS * N_FACTORS)
  return _pmf_sc(users, items, u_flat, v_flat)
